# baseline (speedup 1.0000x reference)
import functools

import jax
import jax.numpy as jnp
from jax import lax
from jax.experimental import pallas as pl
from jax.experimental.pallas import tpu as pltpu
from jax.experimental.pallas import tpu_sc as plsc

_N = 10000
_DE = 16
_NW = 32
_G = 128
_E = 320000
_NG = _E // _G
_GQ = _NG // _NW
_GR = _NG % _NW
_CG = 16
_EC = _CG * _G
_NGP = 2504
_NPAD = 10240
_NPS = _NPAD // 16


def _sc_segment_sum(idx3, attr1):
    mesh = plsc.VectorSubcoreMesh(core_axis_name="c", subcore_axis_name="s")

    @functools.partial(
        pl.kernel,
        mesh=mesh,
        out_type=jax.ShapeDtypeStruct((2, _NPAD // 8, 128), jnp.float32),
        scratch_types=[
            pltpu.VMEM((_GQ + 2, 1, _G), jnp.int32),
            pltpu.VMEM((2, 2 * _CG * 8 * _G), jnp.float32),
            pltpu.VMEM((_EC, _DE), jnp.float32),
            pltpu.VMEM((_NPS // 8, 128), jnp.float32),
            pltpu.VMEM_SHARED((_NPAD, _DE), jnp.float32),
            pltpu.SemaphoreType.DMA,
            pltpu.SemaphoreType.DMA,
            pltpu.SemaphoreType.DMA,
        ],
        compiler_params=pltpu.CompilerParams(use_tc_tiling_on_sc=False,
                                             needs_layout_passes=False),
    )
    def seg_sum(idx_hbm, attr_hbm, out_hbm, idx_v, buf_t, buf16, buf_o, acc,
                lsem0, lsem1, ssem):
        half = 2560000
        cw = _CG * 8 * _G
        c = lax.axis_index("c")
        s = lax.axis_index("s")
        w = s * 2 + c
        ng = _GQ + jnp.where(w < _GR, 1, 0)
        start = _GQ * w + jnp.minimum(w, _GR)
        zvec = jnp.zeros((_DE,), jnp.float32)
        lane = lax.iota(jnp.int32, _DE)
        lsems = [lsem0, lsem1]

        def chunk_base(k):
            if k < 4:
                return (start + k * _CG) * (8 * _G)
            return (start + ng - _CG) * (8 * _G)

        def fire_load(k):
            p = k & 1
            b = chunk_base(k)
            return [
                pltpu.async_copy(attr_hbm.at[pl.ds(b, cw)],
                                 buf_t.at[p, pl.ds(0, cw)], lsems[p]),
                pltpu.async_copy(attr_hbm.at[pl.ds(half + b, cw)],
                                 buf_t.at[p, pl.ds(cw, cw)], lsems[p]),
            ]

        load_h = fire_load(0)

        def zero_body(i, _):
            buf16[i, :] = zvec
            return 0

        lax.fori_loop(0, _NPS, zero_body, 0, unroll=8)
        pltpu.sync_copy(buf16.at[pl.ds(0, _NPS)], acc.at[pl.ds(s * _NPS, _NPS)])
        pltpu.sync_copy(idx_hbm.at[pl.ds(start, 64), pl.ds(0, 1)],
                        idx_v.at[pl.ds(0, 64)])
        pltpu.sync_copy(idx_hbm.at[pl.ds(start + ng - _CG, _CG), pl.ds(0, 1)],
                        idx_v.at[pl.ds(64, _CG)])
        plsc.subcore_barrier()

        def transpose_chunk(p):
            src = buf_t.at[p]
            v16 = jnp.full((_DE,), _DE, jnp.int32)
            v896 = jnp.full((_DE,), 8 * _G - _G, jnp.int32)

            def diag_f(f0, _):
                cv = (lane + f0) & 15
                dl = ((cv >> 3) * cw + (cv & 7) * _G + lane)

                def slab_t(j, carry):
                    def e_body(b, c2):
                        fv, rv = c2
                        x = plsc.load_gather(src, [fv])
                        plsc.store_scatter(buf16, [rv, cv], x)
                        return (fv + v16, rv + v16)

                    fv, rv = lax.fori_loop(0, _G // _DE, e_body, carry,
                                           unroll=8)
                    return (fv + v896, rv)

                lax.fori_loop(0, _CG, slab_t, (dl, lane))
                return 0

            lax.fori_loop(0, _DE, diag_f, 0)

        off = (_CG + 64) - ng

        def scatter_refs(k, j):
            if k < 4:
                return (buf16.at[pl.ds(j * _G, _G)],
                        acc.at[idx_v.at[k * _CG + j, 0]])
            return (buf16.at[pl.ds((off + j) * _G, _G)],
                    acc.at[idx_v.at[64 + off + j, 0]])

        scat_h = []
        for k in range(5):
            if k < 4:
                next_h = fire_load(k + 1)
            for h in load_h:
                h.wait()
            if k < 4:
                load_h = next_h
            for (kk, j, pred) in scat_h:
                if pred is None:
                    srcr, dstr = scatter_refs(kk, j)
                    pltpu.make_async_copy(srcr, dstr, ssem).wait()
            scat_h = []
            transpose_chunk(k & 1)
            if k < 4:
                for j in range(_CG):
                    srcr, dstr = scatter_refs(k, j)
                    pltpu.async_copy(srcr, dstr, ssem, add=True)
                    scat_h.append((k, j, None))
            else:
                for j in range(_CG - 1):
                    @pl.when(64 + j < ng)
                    def _(j=j):
                        srcr, dstr = scatter_refs(4, j)
                        pltpu.async_copy(srcr, dstr, ssem, add=True)

        for j in range(_CG - 1):
            @pl.when(64 + j < ng)
            def _(j=j):
                srcr, dstr = scatter_refs(4, j)
                pltpu.make_async_copy(srcr, dstr, ssem).wait()

        plsc.subcore_barrier()
        pltpu.sync_copy(acc.at[pl.ds(s * _NPS, _NPS)], buf16.at[pl.ds(0, _NPS)])

        def wb_body(i, _):
            for l in range(8):
                buf_o[i, pl.ds(l * _DE, _DE)] = buf16[i * 8 + l, :]
            return 0

        lax.fori_loop(0, _NPS // 8, wb_body, 0)
        pltpu.sync_copy(buf_o,
                        out_hbm.at[c, pl.ds(s * (_NPS // 8), _NPS // 8)])

    return seg_sum(idx3, attr1)


def _tc_mlp_body(nf_ref, p0_ref, p1_ref, w1a_ref, w1b_ref, w2_ref,
                 b1_ref, b2_ref, o_ref):
    agg = p0_ref[...] + p1_ref[...]
    h = jnp.dot(nf_ref[...], w1a_ref[...], preferred_element_type=jnp.float32)
    h = h + jnp.dot(agg, w1b_ref[...], preferred_element_type=jnp.float32)
    h = h + b1_ref[...]
    o = jnp.dot(h, w2_ref[...], preferred_element_type=jnp.float32)
    o_ref[...] = o + b2_ref[...]


def _tc_mlp(node_feats, partials, W1, b1, W2, b2):
    n, d = node_feats.shape
    h_nf = W1.shape[1]
    out_nf = W2.shape[1]
    W1a = W1[:d]
    W1b = W1[d:]
    p0 = partials[0]
    p1 = partials[1]
    blk = 2000
    grid = (n // blk,)
    return pl.pallas_call(
        _tc_mlp_body,
        grid=grid,
        in_specs=[
            pl.BlockSpec((blk, d), lambda i: (i, 0)),
            pl.BlockSpec((blk, _DE), lambda i: (i, 0)),
            pl.BlockSpec((blk, _DE), lambda i: (i, 0)),
            pl.BlockSpec((d, h_nf), lambda i: (0, 0)),
            pl.BlockSpec((_DE, h_nf), lambda i: (0, 0)),
            pl.BlockSpec((h_nf, out_nf), lambda i: (0, 0)),
            pl.BlockSpec((1, h_nf), lambda i: (0, 0)),
            pl.BlockSpec((1, out_nf), lambda i: (0, 0)),
        ],
        out_specs=pl.BlockSpec((blk, out_nf), lambda i: (i, 0)),
        out_shape=jax.ShapeDtypeStruct((n, out_nf), jnp.float32),
    )(node_feats, p0, p1, W1a, W1b, W2,
      b1.reshape(1, h_nf), b2.reshape(1, out_nf))


@jax.jit
def kernel(node_feats, edge_index, edge_attr, W1, b1, W2, b2):
    idx3 = edge_index.reshape(2, _NG, _G).transpose(1, 0, 2)
    attr1 = edge_attr.T.reshape(2, 8, _NG, _G).transpose(0, 2, 1, 3).reshape(-1)
    partials = _sc_segment_sum(idx3, attr1)
    partials = partials.reshape(2, _NPAD, _DE)[:, :_N]
    return _tc_mlp(node_feats, partials, W1, b1, W2, b2)


# speedup vs baseline: 1.4110x; 1.4110x over previous
import functools

import jax
import jax.numpy as jnp
from jax import lax
from jax.experimental import pallas as pl
from jax.experimental.pallas import tpu as pltpu
from jax.experimental.pallas import tpu_sc as plsc

_N = 10000
_DE = 16
_NW = 32
_G = 128
_E = 320000
_NG = _E // _G
_GQ = _NG // _NW
_GR = _NG % _NW
_CG = 16
_EC = _CG * _G
_NGP = 2504
_NPAD = 10240
_NPS = _NPAD // 16


def _sc_segment_sum(idx3, attr1):
    mesh = plsc.VectorSubcoreMesh(core_axis_name="c", subcore_axis_name="s")

    @functools.partial(
        pl.kernel,
        mesh=mesh,
        out_type=jax.ShapeDtypeStruct((2, _NPAD // 8, 128), jnp.float32),
        scratch_types=[
            pltpu.VMEM((_GQ + 2, 1, _G), jnp.int32),
            pltpu.VMEM((2, 2 * _CG * 8 * _G), jnp.float32),
            pltpu.VMEM((_EC, _DE), jnp.float32),
            pltpu.VMEM((_NPS // 8, 128), jnp.float32),
            pltpu.VMEM_SHARED((_NPAD, _DE), jnp.float32),
            pltpu.SemaphoreType.DMA,
            pltpu.SemaphoreType.DMA,
            pltpu.SemaphoreType.DMA,
        ],
        compiler_params=pltpu.CompilerParams(use_tc_tiling_on_sc=False,
                                             needs_layout_passes=False),
    )
    def seg_sum(idx_hbm, attr_hbm, out_hbm, idx_v, buf_t, buf16, buf_o, acc,
                lsem0, lsem1, ssem):
        half = 2560000
        cw = _CG * 8 * _G
        c = lax.axis_index("c")
        s = lax.axis_index("s")
        w = s * 2 + c
        ng = _GQ + jnp.where(w < _GR, 1, 0)
        start = _GQ * w + jnp.minimum(w, _GR)
        zvec = jnp.zeros((_DE,), jnp.float32)
        lane = lax.iota(jnp.int32, _DE)
        lsems = [lsem0, lsem1]

        def chunk_base(k):
            if k < 4:
                return (start + k * _CG) * (8 * _G)
            return (start + ng - _CG) * (8 * _G)

        def fire_load(k):
            p = k & 1
            b = chunk_base(k)
            return [
                pltpu.async_copy(attr_hbm.at[pl.ds(b, cw)],
                                 buf_t.at[p, pl.ds(0, cw)], lsems[p]),
                pltpu.async_copy(attr_hbm.at[pl.ds(half + b, cw)],
                                 buf_t.at[p, pl.ds(cw, cw)], lsems[p]),
            ]

        load_h = fire_load(0)

        def zero_body(i, _):
            buf16[i, :] = zvec
            return 0

        lax.fori_loop(0, _NPS, zero_body, 0, unroll=8)
        pltpu.sync_copy(buf16.at[pl.ds(0, _NPS)], acc.at[pl.ds(s * _NPS, _NPS)])
        pltpu.sync_copy(idx_hbm.at[pl.ds(start, 64), pl.ds(0, 1)],
                        idx_v.at[pl.ds(0, 64)])
        pltpu.sync_copy(idx_hbm.at[pl.ds(start + ng - _CG, _CG), pl.ds(0, 1)],
                        idx_v.at[pl.ds(64, _CG)])
        plsc.subcore_barrier()

        def transpose_chunk(p):
            src = buf_t.at[p]
            v16 = jnp.full((_DE,), _DE, jnp.int32)
            v896 = jnp.full((_DE,), 8 * _G - _G, jnp.int32)

            def diag_f(f0, _):
                cv = (lane + f0) & 15
                dl = ((cv >> 3) * cw + (cv & 7) * _G + lane)

                def slab_t(j, carry):
                    fv, rv = carry
                    xs = [plsc.load_gather(src, [fv + b * _DE])
                          for b in range(_G // _DE)]
                    for b in range(_G // _DE):
                        plsc.store_scatter(buf16, [rv + b * _DE, cv], xs[b])
                    return (fv + v896 + v16 * 8, rv + v16 * 8)

                lax.fori_loop(0, _CG, slab_t, (dl, lane))
                return 0

            lax.fori_loop(0, _DE, diag_f, 0)

        off = (_CG + 64) - ng

        def scatter_refs(k, j):
            if k < 4:
                return (buf16.at[pl.ds(j * _G, _G)],
                        acc.at[idx_v.at[k * _CG + j, 0]])
            return (buf16.at[pl.ds((off + j) * _G, _G)],
                    acc.at[idx_v.at[64 + off + j, 0]])

        scat_h = []
        for k in range(5):
            if k < 4:
                next_h = fire_load(k + 1)
            for h in load_h:
                h.wait()
            if k < 4:
                load_h = next_h
            for (kk, j, pred) in scat_h:
                if pred is None:
                    srcr, dstr = scatter_refs(kk, j)
                    pltpu.make_async_copy(srcr, dstr, ssem).wait()
            scat_h = []
            transpose_chunk(k & 1)
            if k < 4:
                for j in range(_CG):
                    srcr, dstr = scatter_refs(k, j)
                    pltpu.async_copy(srcr, dstr, ssem, add=True)
                    scat_h.append((k, j, None))
            else:
                for j in range(_CG - 1):
                    @pl.when(64 + j < ng)
                    def _(j=j):
                        srcr, dstr = scatter_refs(4, j)
                        pltpu.async_copy(srcr, dstr, ssem, add=True)

        for j in range(_CG - 1):
            @pl.when(64 + j < ng)
            def _(j=j):
                srcr, dstr = scatter_refs(4, j)
                pltpu.make_async_copy(srcr, dstr, ssem).wait()

        plsc.subcore_barrier()
        pltpu.sync_copy(acc.at[pl.ds(s * _NPS, _NPS)], buf16.at[pl.ds(0, _NPS)])

        def wb_body(i, _):
            for l in range(8):
                buf_o[i, pl.ds(l * _DE, _DE)] = buf16[i * 8 + l, :]
            return 0

        lax.fori_loop(0, _NPS // 8, wb_body, 0)
        pltpu.sync_copy(buf_o,
                        out_hbm.at[c, pl.ds(s * (_NPS // 8), _NPS // 8)])

    return seg_sum(idx3, attr1)


def _tc_mlp_body(nf_ref, p0_ref, p1_ref, w1a_ref, w1b_ref, w2_ref,
                 b1_ref, b2_ref, o_ref):
    agg = p0_ref[...] + p1_ref[...]
    h = jnp.dot(nf_ref[...], w1a_ref[...], preferred_element_type=jnp.float32)
    h = h + jnp.dot(agg, w1b_ref[...], preferred_element_type=jnp.float32)
    h = h + b1_ref[...]
    o = jnp.dot(h, w2_ref[...], preferred_element_type=jnp.float32)
    o_ref[...] = o + b2_ref[...]


def _tc_mlp(node_feats, partials, W1, b1, W2, b2):
    n, d = node_feats.shape
    h_nf = W1.shape[1]
    out_nf = W2.shape[1]
    W1a = W1[:d]
    W1b = W1[d:]
    p0 = partials[0]
    p1 = partials[1]
    blk = 2000
    grid = (n // blk,)
    return pl.pallas_call(
        _tc_mlp_body,
        grid=grid,
        in_specs=[
            pl.BlockSpec((blk, d), lambda i: (i, 0)),
            pl.BlockSpec((blk, _DE), lambda i: (i, 0)),
            pl.BlockSpec((blk, _DE), lambda i: (i, 0)),
            pl.BlockSpec((d, h_nf), lambda i: (0, 0)),
            pl.BlockSpec((_DE, h_nf), lambda i: (0, 0)),
            pl.BlockSpec((h_nf, out_nf), lambda i: (0, 0)),
            pl.BlockSpec((1, h_nf), lambda i: (0, 0)),
            pl.BlockSpec((1, out_nf), lambda i: (0, 0)),
        ],
        out_specs=pl.BlockSpec((blk, out_nf), lambda i: (i, 0)),
        out_shape=jax.ShapeDtypeStruct((n, out_nf), jnp.float32),
    )(node_feats, p0, p1, W1a, W1b, W2,
      b1.reshape(1, h_nf), b2.reshape(1, out_nf))


@jax.jit
def kernel(node_feats, edge_index, edge_attr, W1, b1, W2, b2):
    idx3 = edge_index.reshape(2, _NG, _G).transpose(1, 0, 2)
    attr1 = edge_attr.T.reshape(2, 8, _NG, _G).transpose(0, 2, 1, 3).reshape(-1)
    partials = _sc_segment_sum(idx3, attr1)
    partials = partials.reshape(2, _NPAD, _DE)[:, :_N]
    return _tc_mlp(node_feats, partials, W1, b1, W2, b2)
